# Initial kernel scaffold; baseline (speedup 1.0000x reference)
import jax
import jax.numpy as jnp
from jax.experimental import pallas as pl


def kernel(x_1st, x_2nd, edge, A_masked, W_node, b_node):
    raise NotImplementedError("write your pallas kernel here")


# speedup vs baseline: 5.1973x; 5.1973x over previous
import functools

import jax
import jax.numpy as jnp
from jax import lax
from jax.experimental import pallas as pl
from jax.experimental.pallas import tpu as pltpu
from jax.experimental.pallas import tpu_sc as plsc

T = 10000
D = 128
M = 160000
NT = 16
NC = 2
EPT = M // NT
CH = 128
NCHUNK = (EPT + CH - 1) // CH
EPAD = NCHUNK * CH
SLAB = 624


def _mp_body(xt_hbm, no_hbm, ni_hbm, a_hbm, out_hbm,
             no_v, ni_v, a_v, rows_v, acc_sh, sem):
    c = lax.axis_index("c")
    s = lax.axis_index("s")

    pltpu.sync_copy(no_hbm.at[s], no_v)
    pltpu.sync_copy(ni_hbm.at[s], ni_v)
    pltpu.sync_copy(a_hbm.at[s], a_v)

    coff = c * T

    def _off(i, _):
        v = no_v[pl.ds(i * 16, 16)]
        no_v[pl.ds(i * 16, 16)] = v + coff
        return 0

    lax.fori_loop(0, EPAD // 16, _off, 0)

    pltpu.sync_copy(xt_hbm.at[pl.ds(c * T + s * SLAB, SLAB)],
                    acc_sh.at[pl.ds(s * SLAB, SLAB)])

    @pl.when(s == NT - 1)
    def _init_tail():
        pltpu.sync_copy(xt_hbm.at[pl.ds(c * T + NT * SLAB, T - NT * SLAB)],
                        acc_sh.at[pl.ds(NT * SLAB, T - NT * SLAB)])

    plsc.subcore_barrier()

    def _chunk(k, _):
        pltpu.async_copy(xt_hbm.at[no_v.at[pl.ds(k * CH, CH)]], rows_v,
                         sem).wait()

        def _grp(g, _):
            a_vec = a_v[pl.ds(k * CH + g * 16, 16)]
            for i in range(16):
                e = g * 16 + i
                ab = jnp.full((16,), a_vec[i])
                for jj in range(D // 16):
                    rv = rows_v[e, pl.ds(jj * 16, 16)]
                    rows_v[e, pl.ds(jj * 16, 16)] = rv * ab
            return 0

        lax.fori_loop(0, CH // 16, _grp, 0)

        pltpu.sync_copy(rows_v, acc_sh.at[ni_v.at[pl.ds(k * CH, CH)]],
                        add=True)
        return 0

    lax.fori_loop(0, NCHUNK, _chunk, 0)
    plsc.subcore_barrier()

    pltpu.sync_copy(acc_sh.at[pl.ds(s * SLAB, SLAB)],
                    out_hbm.at[c, pl.ds(s * SLAB, SLAB)])

    @pl.when(s == NT - 1)
    def _out_tail():
        pltpu.sync_copy(acc_sh.at[pl.ds(NT * SLAB, T - NT * SLAB)],
                        out_hbm.at[c, pl.ds(NT * SLAB, T - NT * SLAB)])


_mp_kernel = functools.partial(
    pl.kernel,
    out_type=jax.ShapeDtypeStruct((NC, T, D), jnp.float32),
    mesh=plsc.VectorSubcoreMesh(core_axis_name="c", subcore_axis_name="s"),
    scratch_types=[
        pltpu.VMEM((EPAD,), jnp.int32),
        pltpu.VMEM((EPAD,), jnp.int32),
        pltpu.VMEM((EPAD,), jnp.float32),
        pltpu.VMEM((CH, D), jnp.float32),
        pltpu.VMEM_SHARED((T, D), jnp.float32),
        pltpu.SemaphoreType.DMA,
    ],
)(_mp_body)


BT = 400


def _mm_body(u_ref, wt_ref, b_ref, o_ref):
    wt = wt_ref[...]
    b = b_ref[...]
    y0 = jnp.dot(u_ref[0], wt, preferred_element_type=jnp.float32) + b
    y1 = jnp.dot(u_ref[1], wt, preferred_element_type=jnp.float32) + b
    o_ref[...] = jnp.stack([y0, y1], axis=1)


def _linear(upd, wt, b2):
    return pl.pallas_call(
        _mm_body,
        grid=(T // BT,),
        in_specs=[
            pl.BlockSpec((NC, BT, D), lambda t: (0, t, 0)),
            pl.BlockSpec((D, D), lambda t: (0, 0)),
            pl.BlockSpec((1, D), lambda t: (0, 0)),
        ],
        out_specs=pl.BlockSpec((BT, NC, D), lambda t: (t, 0, 0)),
        out_shape=jax.ShapeDtypeStruct((T, NC, D), jnp.float32),
    )(upd, wt, b2)


def kernel(x_1st, x_2nd, edge, A_masked, W_node, b_node):
    xt = jnp.transpose(x_1st[0], (1, 0, 2)).reshape(NC * T, D)
    pad = ((0, 0), (0, EPAD - EPT))
    ni = jnp.pad(edge[0].reshape(NT, EPT), pad)
    no = jnp.pad(edge[1].reshape(NT, EPT), pad)
    a = jnp.pad(A_masked[0, 0].reshape(NT, EPT), pad)

    upd = _mp_kernel(xt, no, ni, a)

    out = _linear(upd, W_node.T, b_node[None, :])
    return out.reshape(1, T, NC, D)
